# baseline (device time: 240766 ns/iter reference)
import jax
import jax.numpy as jnp
from jax import lax
from jax.experimental import pallas as pl
from jax.experimental.pallas import tpu as pltpu

N_DEV = 32
M = 2048
N = 2048
CH = M // N_DEV


def kernel(x, w_mat):
    def body(x_ref, w_ref, out_ref, acc_ref, comm_ref, send_sems, recv_sems):
        my = lax.axis_index("i")
        left = jax.lax.rem(my + N_DEV - 1, N_DEV)
        right = jax.lax.rem(my + 1, N_DEV)

        barrier_sem = pltpu.get_barrier_semaphore()
        for nbr in (left, right):
            pl.semaphore_signal(
                barrier_sem, inc=1,
                device_id=(nbr,), device_id_type=pl.DeviceIdType.MESH,
            )
        pl.semaphore_wait(barrier_sem, 2)

        acc_ref[:, :] = jnp.dot(
            x_ref[:, :], w_ref[:, :], preferred_element_type=jnp.float32
        )

        for s in range(N_DEV - 1):
            c_send = jax.lax.rem(my + (N_DEV - 1 - s), N_DEV)
            rdma = pltpu.make_async_remote_copy(
                src_ref=acc_ref.at[pl.ds(c_send * CH, CH), :],
                dst_ref=comm_ref.at[s],
                send_sem=send_sems.at[s],
                recv_sem=recv_sems.at[s],
                device_id=(right,),
                device_id_type=pl.DeviceIdType.MESH,
            )
            rdma.start()
            rdma.wait()

            c_recv = jax.lax.rem(my + (N_DEV - 2 - s), N_DEV)
            acc_ref[pl.ds(c_recv * CH, CH), :] = (
                acc_ref[pl.ds(c_recv * CH, CH), :] + comm_ref[s, :, :]
            )

        out_ref[:, :] = acc_ref[pl.ds(my * CH, CH), :]

    return pl.pallas_call(
        body,
        out_shape=jax.ShapeDtypeStruct((CH, N), jnp.float32),
        in_specs=[
            pl.BlockSpec(memory_space=pltpu.VMEM),
            pl.BlockSpec(memory_space=pltpu.VMEM),
        ],
        out_specs=pl.BlockSpec(memory_space=pltpu.VMEM),
        scratch_shapes=[
            pltpu.VMEM((M, N), jnp.float32),
            pltpu.VMEM((N_DEV - 1, CH, N), jnp.float32),
            pltpu.SemaphoreType.DMA((N_DEV - 1,)),
            pltpu.SemaphoreType.DMA((N_DEV - 1,)),
        ],
        compiler_params=pltpu.CompilerParams(collective_id=0),
    )(x, w_mat)


# device time: 191115 ns/iter; 1.2598x vs baseline; 1.2598x over previous
import jax
import jax.numpy as jnp
from jax import lax
from jax.experimental import pallas as pl
from jax.experimental.pallas import tpu as pltpu

N_DEV = 32
M = 2048
N = 2048
CH = M // N_DEV
K = 1
CW = (N // 2) // K


def kernel(x, w_mat):
    def body(x_ref, w_ref, out_ref, acc_ref, comm_ref, send_sems, recv_sems):
        my = lax.axis_index("i")
        left = lax.rem(my + N_DEV - 1, N_DEV)
        right = lax.rem(my + 1, N_DEV)

        barrier_sem = pltpu.get_barrier_semaphore()
        for nbr in (left, right):
            pl.semaphore_signal(
                barrier_sem, inc=1,
                device_id=(nbr,), device_id_type=pl.DeviceIdType.MESH,
            )
        pl.semaphore_wait(barrier_sem, 2)

        acc_ref[:, :] = jnp.dot(
            x_ref[:, :], w_ref[:, :], preferred_element_type=jnp.float32
        )

        def send_chunk(r, s):
            if r == 0:
                return lax.rem(my + (N_DEV - 1 - s), N_DEV)
            return lax.rem(my + s + 1, N_DEV)

        def recv_chunk(r, s):
            if r == 0:
                return lax.rem(my + (N_DEV - 2 - s), N_DEV)
            return lax.rem(my + s + 2, N_DEV)

        rings = [(r, j) for r in (0, 1) for j in range(K)]

        def col_ds(r, j):
            return pl.ds((r * K + j) * CW, CW)

        def start_send(r, j, s):
            rdma = pltpu.make_async_remote_copy(
                src_ref=acc_ref.at[pl.ds(send_chunk(r, s) * CH, CH), col_ds(r, j)],
                dst_ref=comm_ref.at[r, j, s],
                send_sem=send_sems.at[r, j, s],
                recv_sem=recv_sems.at[r, j, s],
                device_id=(right if r == 0 else left,),
                device_id_type=pl.DeviceIdType.MESH,
            )
            rdma.start()
            return rdma

        rdmas = {}
        for r, j in rings:
            rdmas[(r, j, 0)] = start_send(r, j, 0)

        for s in range(N_DEV - 1):
            for r, j in rings:
                rdmas[(r, j, s)].wait_recv()
                rows = pl.ds(recv_chunk(r, s) * CH, CH)
                acc_ref[rows, col_ds(r, j)] = (
                    acc_ref[rows, col_ds(r, j)] + comm_ref[r, j, s, :, :]
                )
                if s < N_DEV - 2:
                    rdmas[(r, j, s + 1)] = start_send(r, j, s + 1)

        for (r, j, s), rdma in rdmas.items():
            rdma.wait_send()

        out_ref[:, :] = acc_ref[pl.ds(my * CH, CH), :]

    return pl.pallas_call(
        body,
        out_shape=jax.ShapeDtypeStruct((CH, N), jnp.float32),
        in_specs=[
            pl.BlockSpec(memory_space=pltpu.VMEM),
            pl.BlockSpec(memory_space=pltpu.VMEM),
        ],
        out_specs=pl.BlockSpec(memory_space=pltpu.VMEM),
        scratch_shapes=[
            pltpu.VMEM((M, N), jnp.float32),
            pltpu.VMEM((2, K, N_DEV - 1, CH, CW), jnp.float32),
            pltpu.SemaphoreType.DMA((2, K, N_DEV - 1)),
            pltpu.SemaphoreType.DMA((2, K, N_DEV - 1)),
        ],
        compiler_params=pltpu.CompilerParams(collective_id=0),
    )(x, w_mat)
